# baseline (speedup 1.0000x reference)
import functools

import jax
import jax.numpy as jnp
from jax import lax
from jax.experimental import pallas as pl
from jax.experimental.pallas import tpu as pltpu
from jax.experimental.pallas import tpu_sc as plsc

_F32 = jnp.float32
_EPS = 1e-5
_NUM_GRAPHS = 64
_NBUF = 8
_CHUNK = 32
_LOOK = 6


@functools.lru_cache(maxsize=None)
def _make_prop(n, npad, npack, d):
    mesh = plsc.VectorSubcoreMesh(core_axis_name="c", subcore_axis_name="s")
    zrows = npad // 16
    nchunk = npack * 128 // _CHUNK

    @functools.partial(
        pl.kernel,
        out_type=jax.ShapeDtypeStruct((2 * npad, d), _F32),
        mesh=mesh,
        scratch_types=[
            [pltpu.VMEM((_CHUNK, d), _F32) for _ in range(_NBUF)],
            pltpu.VMEM((npack, 128), jnp.int32),
            pltpu.VMEM((_NBUF, _CHUNK), jnp.int32),
            pltpu.VMEM((_NBUF, _CHUNK), jnp.int32),
            pltpu.VMEM_SHARED((npad, d), _F32),
            [pltpu.SemaphoreType.DMA] * _NBUF,
            [pltpu.SemaphoreType.DMA] * _NBUF,
        ],
    )
    def prop(hw_hbm, packed_hbm, zero_hbm, out_hbm,
             rows_v, packed_v, srcb, dstb, acc, gsems, ssems):
        c = lax.axis_index("c")
        s = lax.axis_index("s")
        wid = c * 16 + s
        pltpu.sync_copy(zero_hbm.at[pl.ds(s * zrows, zrows)],
                        acc.at[pl.ds(s * zrows, zrows)])
        pltpu.sync_copy(packed_hbm.at[wid], packed_v)

        def unpack(j, b):
            row = j // (128 // _CHUNK)
            base = (j % (128 // _CHUNK)) * _CHUNK
            for k in range(_CHUNK // 16):
                pk = packed_v[row, pl.ds(base + k * 16, 16)]
                srcb[b, pl.ds(k * 16, 16)] = lax.shift_right_logical(pk, 16)
                dstb[b, pl.ds(k * 16, 16)] = pk & 0xFFFF

        def gather(j, b):
            pltpu.async_copy(hw_hbm.at[srcb.at[b]], rows_v[b], gsems[b])

        def gwait(b):
            pltpu.make_async_copy(hw_hbm.at[srcb.at[b]],
                                  rows_v[b], gsems[b]).wait()

        def scatter(b):
            pltpu.async_copy(rows_v[b], acc.at[dstb.at[b]], ssems[b],
                             add=True)

        def swait(b):
            pltpu.make_async_copy(rows_v[b], acc.at[dstb.at[b]],
                                  ssems[b]).wait()

        for b in range(_LOOK):
            unpack(b, b)
            gather(b, b)
        plsc.subcore_barrier()
        for j in range(_NBUF):
            b = j % _NBUF
            bb = (j + _LOOK) % _NBUF
            if j >= _NBUF - _LOOK:
                swait(bb)
            unpack(j + _LOOK, bb)
            gather(j + _LOOK, bb)
            gwait(b)
            scatter(b)

        @pl.loop(_NBUF, nchunk, step=_NBUF)
        def _(j0):
            for i in range(_NBUF):
                j = j0 + i
                b = i
                bb = (i + _LOOK) % _NBUF
                swait(bb)

                @pl.when(j + _LOOK < nchunk)
                def _():
                    unpack(j + _LOOK, bb)
                    gather(j + _LOOK, bb)

                gwait(b)
                scatter(b)

        for t in range(_NBUF - _LOOK):
            swait((nchunk - (_NBUF - _LOOK) + t) % _NBUF)
        plsc.subcore_barrier()
        pltpu.sync_copy(
            acc.at[pl.ds(s * zrows, zrows)],
            out_hbm.at[pl.ds(c * npad + s * zrows, zrows)])

    return prop


def _dot(a, b):
    return jnp.dot(a, b, preferred_element_type=_F32,
                   precision=jax.lax.Precision.HIGHEST)


@functools.lru_cache(maxsize=None)
def _make_tc_first(n, npad):
    def body(degp_ref, x_ref, w_ref, hw_ref, dinv_ref):
        deg = degp_ref[:n, :1] + degp_ref[npad:npad + n, :1]
        dinv = jnp.where(deg > 0, jax.lax.rsqrt(deg), 0.0)
        dinv_ref[...] = dinv
        hw_ref[...] = _dot(x_ref[...], w_ref[...]) * dinv
    return body


@functools.lru_cache(maxsize=None)
def _make_tc_mid(n, npad):
    def body(p_ref, dinv_ref, b_ref, g_ref, bt_ref, w_ref, out_ref):
        s = p_ref[:n] + p_ref[npad:npad + n]
        dinv = dinv_ref[...]
        pre = s * dinv + b_ref[...]
        m = jnp.mean(pre, axis=0, keepdims=True)
        cen = pre - m
        v = jnp.mean(cen * cen, axis=0, keepdims=True)
        h = jnp.maximum(cen * jax.lax.rsqrt(v + _EPS) * g_ref[...]
                        + bt_ref[...], 0.0)
        out_ref[...] = _dot(h, w_ref[...]) * dinv
    return body


@functools.lru_cache(maxsize=None)
def _make_tc_bottleneck(n, npad, g):
    def body(p_ref, dinv_ref, b_ref, gm_ref, bt_ref, brow_ref, bcol_ref,
             wl1_ref, bl1_ref, wl2_ref, bl2_ref,
             v1_ref, c1_ref, v2_ref, c2_ref, wd_ref,
             out_ref, emb_ref):
        s = p_ref[:n] + p_ref[npad:npad + n]
        dinv = dinv_ref[...]
        pre = s * dinv + b_ref[...]
        m = jnp.mean(pre, axis=0, keepdims=True)
        cen = pre - m
        v = jnp.mean(cen * cen, axis=0, keepdims=True)
        h = jnp.maximum(cen * jax.lax.rsqrt(v + _EPS) * gm_ref[...]
                        + bt_ref[...], 0.0)
        iota_gn = lax.broadcasted_iota(jnp.int32, (g, n), 0)
        onehot_gn = (iota_gn == brow_ref[...]).astype(_F32)
        sums = _dot(onehot_gn, h)
        cnts = jnp.sum(onehot_gn, axis=1, keepdims=True)
        pooled = sums / jnp.maximum(cnts, 1.0)
        e1 = jnp.maximum(_dot(pooled, wl1_ref[...]) + bl1_ref[...], 0.0)
        emb = _dot(e1, wl2_ref[...]) + bl2_ref[...]
        emb_ref[...] = emb
        d1 = jnp.maximum(_dot(emb, v1_ref[...]) + c1_ref[...], 0.0)
        d2 = jnp.maximum(_dot(d1, v2_ref[...]) + c2_ref[...], 0.0)
        dw = _dot(d2, wd_ref[...])
        iota_ng = lax.broadcasted_iota(jnp.int32, (n, g), 1)
        onehot_ng = (iota_ng == bcol_ref[...]).astype(_F32)
        out_ref[...] = _dot(onehot_ng, dw) * dinv
    return body


@functools.lru_cache(maxsize=None)
def _make_tc_last(n, npad):
    def body(p_ref, dinv_ref, b_ref, out_ref):
        s = p_ref[:n] + p_ref[npad:npad + n]
        out_ref[...] = jax.nn.sigmoid(s * dinv_ref[...] + b_ref[...])
    return body


def _tc(body, out_shapes, *args):
    return pl.pallas_call(body, out_shape=out_shapes)(*args)


def kernel(x, edge_index, batch_index, params):
    n, d_in = x.shape
    e = edge_index.shape[1]
    hid = params["enc_convs"][0][0].shape[1]
    g = _NUM_GRAPHS

    e2 = e + n
    ept = -(-e2 // (32 * 1024)) * 1024
    epad = 32 * ept
    npack = ept // 128
    npad = -(-(n + 1) // 128) * 128

    pad = epad - e2
    loop = jnp.arange(n, dtype=jnp.int32)
    src = jnp.concatenate([edge_index[0].astype(jnp.int32), loop,
                           jnp.full((pad,), n - 1, jnp.int32)])
    dst = jnp.concatenate([edge_index[1].astype(jnp.int32), loop,
                           n + jnp.arange(pad, dtype=jnp.int32) % (npad - n)])
    packed = jnp.sort((src << 16) | dst)
    packed_p = packed.reshape(32, npack, 128)
    zeros_d = jnp.zeros((npad, hid), _F32)
    ones_n = jnp.ones((n, hid), _F32)
    brow = batch_index.astype(jnp.int32).reshape(1, n)
    bcol = batch_index.astype(jnp.int32).reshape(n, 1)

    prop = _make_prop(n, npad, npack, hid)
    f32 = jax.ShapeDtypeStruct

    def r1(v):
        return v.reshape(1, -1)

    degp = prop(ones_n, packed_p, zeros_d)
    w0 = params["enc_convs"][0][0]
    hw, dinv = _tc(_make_tc_first(n, npad),
                   [f32((n, hid), _F32), f32((n, 1), _F32)], degp, x, w0)

    for i in range(3):
        partials = prop(hw, packed_p, zeros_d)
        b_i = params["enc_convs"][i][1]
        gm, bt = params["enc_bns"][i]
        w_next = params["enc_convs"][i + 1][0]
        hw = _tc(_make_tc_mid(n, npad), f32((n, hid), _F32),
                 partials, dinv, r1(b_i), r1(gm), r1(bt), w_next)

    partials = prop(hw, packed_p, zeros_d)
    b3 = params["enc_convs"][3][1]
    gm3, bt3 = params["enc_bns"][3]
    (wl1, bl1), (wl2, bl2) = params["fc_latent"]
    (v1, c1), (v2, c2) = params["fc_decoder"]
    wd0 = params["dec_convs"][0][0]
    emb_dim = wl2.shape[1]
    hw, emb = _tc(_make_tc_bottleneck(n, npad, g),
                  [f32((n, hid), _F32), f32((g, emb_dim), _F32)],
                  partials, dinv, r1(b3), r1(gm3), r1(bt3), brow, bcol,
                  wl1, r1(bl1), wl2, r1(bl2), v1, r1(c1), v2, r1(c2), wd0)

    for i in range(3):
        partials = prop(hw, packed_p, zeros_d)
        b_i = params["dec_convs"][i][1]
        gm, bt = params["dec_bns"][i]
        w_next = params["dec_convs"][i + 1][0]
        hw = _tc(_make_tc_mid(n, npad), f32((n, hid), _F32),
                 partials, dinv, r1(b_i), r1(gm), r1(bt), w_next)

    partials = prop(hw, packed_p, zeros_d)
    b_last = params["dec_convs"][3][1]
    h_out = _tc(_make_tc_last(n, npad), f32((n, d_in), _F32),
                partials, dinv, r1(b_last))
    return (h_out, emb)


# speedup vs baseline: 1.1046x; 1.1046x over previous
import functools

import jax
import jax.numpy as jnp
from jax import lax
from jax.experimental import pallas as pl
from jax.experimental.pallas import tpu as pltpu
from jax.experimental.pallas import tpu_sc as plsc

_F32 = jnp.float32
_EPS = 1e-5
_NUM_GRAPHS = 64
_NBUF = 8
_CHUNK = 32
_LOOK = 6


@functools.lru_cache(maxsize=None)
def _make_prop(n, npad, npack, d):
    mesh = plsc.VectorSubcoreMesh(core_axis_name="c", subcore_axis_name="s")
    zrows = npad // 16
    nchunk = npack * 128 // _CHUNK

    @functools.partial(
        pl.kernel,
        out_type=jax.ShapeDtypeStruct((2 * npad, d), _F32),
        mesh=mesh,
        scratch_types=[
            [pltpu.VMEM((_CHUNK, d), _F32) for _ in range(_NBUF)],
            pltpu.VMEM((npack, 128), jnp.int32),
            pltpu.VMEM((_NBUF, _CHUNK), jnp.int32),
            pltpu.VMEM((_NBUF, _CHUNK), jnp.int32),
            pltpu.VMEM_SHARED((npad, d), _F32),
            [pltpu.SemaphoreType.DMA] * _NBUF,
            [pltpu.SemaphoreType.DMA] * _NBUF,
        ],
    )
    def prop(hw_hbm, packed_hbm, zero_hbm, out_hbm,
             rows_v, packed_v, srcb, dstb, acc, gsems, ssems):
        c = lax.axis_index("c")
        s = lax.axis_index("s")
        wid = c * 16 + s
        pltpu.sync_copy(zero_hbm.at[pl.ds(s * zrows, zrows)],
                        acc.at[pl.ds(s * zrows, zrows)])
        pltpu.sync_copy(packed_hbm.at[wid], packed_v)

        def unpack(j, b):
            row = j // (128 // _CHUNK)
            base = (j % (128 // _CHUNK)) * _CHUNK
            for k in range(_CHUNK // 16):
                pk = packed_v[row, pl.ds(base + k * 16, 16)]
                srcb[b, pl.ds(k * 16, 16)] = pk & 0xFFFF
                dstb[b, pl.ds(k * 16, 16)] = lax.shift_right_logical(pk, 16)

        def gather(j, b):
            pltpu.async_copy(hw_hbm.at[srcb.at[b]], rows_v[b], gsems[b])

        def gwait(b):
            pltpu.make_async_copy(hw_hbm.at[srcb.at[b]],
                                  rows_v[b], gsems[b]).wait()

        def scatter(b):
            pltpu.async_copy(rows_v[b], acc.at[dstb.at[b]], ssems[b],
                             add=True)

        def swait(b):
            pltpu.make_async_copy(rows_v[b], acc.at[dstb.at[b]],
                                  ssems[b]).wait()

        for b in range(_LOOK):
            unpack(b, b)
            gather(b, b)
        plsc.subcore_barrier()
        for j in range(_NBUF):
            b = j % _NBUF
            bb = (j + _LOOK) % _NBUF
            if j >= _NBUF - _LOOK:
                swait(bb)
            unpack(j + _LOOK, bb)
            gather(j + _LOOK, bb)
            gwait(b)
            scatter(b)

        @pl.loop(_NBUF, nchunk, step=_NBUF)
        def _(j0):
            for i in range(_NBUF):
                j = j0 + i
                b = i
                bb = (i + _LOOK) % _NBUF
                swait(bb)

                @pl.when(j + _LOOK < nchunk)
                def _():
                    unpack(j + _LOOK, bb)
                    gather(j + _LOOK, bb)

                gwait(b)
                scatter(b)

        for t in range(_NBUF - _LOOK):
            swait((nchunk - (_NBUF - _LOOK) + t) % _NBUF)
        plsc.subcore_barrier()
        pltpu.sync_copy(
            acc.at[pl.ds(s * zrows, zrows)],
            out_hbm.at[pl.ds(c * npad + s * zrows, zrows)])

    return prop


def _dot(a, b):
    return jnp.dot(a, b, preferred_element_type=_F32,
                   precision=jax.lax.Precision.HIGHEST)


@functools.lru_cache(maxsize=None)
def _make_tc_first(n, npad):
    def body(deg_ref, x_ref, w_ref, hw_ref, dinv_ref):
        deg = deg_ref[...]
        dinv = jnp.where(deg > 0, jax.lax.rsqrt(deg), 0.0)
        dinv_ref[...] = dinv
        hw_ref[...] = _dot(x_ref[...], w_ref[...]) * dinv
    return body


@functools.lru_cache(maxsize=None)
def _make_tc_deg(nhi, blk):
    def body(dr_ref, dc_ref, out_ref):
        i = pl.program_id(0)

        @pl.when(i == 0)
        def _():
            out_ref[...] = jnp.zeros_like(out_ref)

        hi_r = dr_ref[...] // 128
        lo_c = dc_ref[...] % 128
        oh_hi = (lax.broadcasted_iota(jnp.int32, (nhi, blk), 0)
                 == hi_r).astype(_F32)
        oh_lo = (lax.broadcasted_iota(jnp.int32, (blk, 128), 1)
                 == lo_c).astype(_F32)
        out_ref[...] += _dot(oh_hi, oh_lo)
    return body


@functools.lru_cache(maxsize=None)
def _make_tc_mid(n, npad):
    def body(p_ref, dinv_ref, b_ref, g_ref, bt_ref, w_ref, out_ref):
        s = p_ref[:n] + p_ref[npad:npad + n]
        dinv = dinv_ref[...]
        pre = s * dinv + b_ref[...]
        m = jnp.mean(pre, axis=0, keepdims=True)
        cen = pre - m
        v = jnp.mean(cen * cen, axis=0, keepdims=True)
        h = jnp.maximum(cen * jax.lax.rsqrt(v + _EPS) * g_ref[...]
                        + bt_ref[...], 0.0)
        out_ref[...] = _dot(h, w_ref[...]) * dinv
    return body


@functools.lru_cache(maxsize=None)
def _make_tc_bottleneck(n, npad, g):
    def body(p_ref, dinv_ref, b_ref, gm_ref, bt_ref, brow_ref, bcol_ref,
             wl1_ref, bl1_ref, wl2_ref, bl2_ref,
             v1_ref, c1_ref, v2_ref, c2_ref, wd_ref,
             out_ref, emb_ref):
        s = p_ref[:n] + p_ref[npad:npad + n]
        dinv = dinv_ref[...]
        pre = s * dinv + b_ref[...]
        m = jnp.mean(pre, axis=0, keepdims=True)
        cen = pre - m
        v = jnp.mean(cen * cen, axis=0, keepdims=True)
        h = jnp.maximum(cen * jax.lax.rsqrt(v + _EPS) * gm_ref[...]
                        + bt_ref[...], 0.0)
        iota_gn = lax.broadcasted_iota(jnp.int32, (g, n), 0)
        onehot_gn = (iota_gn == brow_ref[...]).astype(_F32)
        sums = _dot(onehot_gn, h)
        cnts = jnp.sum(onehot_gn, axis=1, keepdims=True)
        pooled = sums / jnp.maximum(cnts, 1.0)
        e1 = jnp.maximum(_dot(pooled, wl1_ref[...]) + bl1_ref[...], 0.0)
        emb = _dot(e1, wl2_ref[...]) + bl2_ref[...]
        emb_ref[...] = emb
        d1 = jnp.maximum(_dot(emb, v1_ref[...]) + c1_ref[...], 0.0)
        d2 = jnp.maximum(_dot(d1, v2_ref[...]) + c2_ref[...], 0.0)
        dw = _dot(d2, wd_ref[...])
        iota_ng = lax.broadcasted_iota(jnp.int32, (n, g), 1)
        onehot_ng = (iota_ng == bcol_ref[...]).astype(_F32)
        out_ref[...] = _dot(onehot_ng, dw) * dinv
    return body


@functools.lru_cache(maxsize=None)
def _make_tc_last(n, npad):
    def body(p_ref, dinv_ref, b_ref, out_ref):
        s = p_ref[:n] + p_ref[npad:npad + n]
        out_ref[...] = jax.nn.sigmoid(s * dinv_ref[...] + b_ref[...])
    return body


def _tc(body, out_shapes, *args):
    return pl.pallas_call(body, out_shape=out_shapes)(*args)


def _tc_grid(body, grid, in_specs, out_specs, out_shape, *args):
    return pl.pallas_call(body, grid=grid, in_specs=in_specs,
                          out_specs=out_specs, out_shape=out_shape)(*args)


def kernel(x, edge_index, batch_index, params):
    n, d_in = x.shape
    e = edge_index.shape[1]
    hid = params["enc_convs"][0][0].shape[1]
    g = _NUM_GRAPHS

    e2 = e + n
    ept = -(-e2 // (32 * 1024)) * 1024
    epad = 32 * ept
    npack = ept // 128
    npad = -(-(n + 1) // 128) * 128

    pad = epad - e2
    loop = jnp.arange(n, dtype=jnp.int32)
    src = jnp.concatenate([edge_index[0].astype(jnp.int32), loop,
                           jnp.zeros((pad,), jnp.int32)])
    dst = jnp.concatenate([edge_index[1].astype(jnp.int32), loop,
                           n + jnp.arange(pad, dtype=jnp.int32) % (npad - n)])
    packed_p = (src | (dst << 16)).reshape(32, npack, 128)
    zeros_d = jnp.zeros((npad, hid), _F32)
    brow = batch_index.astype(jnp.int32).reshape(1, n)
    bcol = batch_index.astype(jnp.int32).reshape(n, 1)

    prop = _make_prop(n, npad, npack, hid)
    f32 = jax.ShapeDtypeStruct

    def r1(v):
        return v.reshape(1, -1)

    blk = 8192
    nhi = npad // 128
    dst_row = dst.reshape(1, epad)
    dst_col = dst.reshape(epad, 1)
    deg_tab = _tc_grid(
        _make_tc_deg(nhi, blk), (epad // blk,),
        [pl.BlockSpec((1, blk), lambda i: (0, i)),
         pl.BlockSpec((blk, 1), lambda i: (i, 0))],
        pl.BlockSpec((nhi, 128), lambda i: (0, 0)),
        f32((nhi, 128), _F32), dst_row, dst_col)
    deg_vec = deg_tab.reshape(-1)[:n].reshape(n, 1)
    w0 = params["enc_convs"][0][0]
    hw, dinv = _tc(_make_tc_first(n, npad),
                   [f32((n, hid), _F32), f32((n, 1), _F32)], deg_vec, x, w0)

    for i in range(3):
        partials = prop(hw, packed_p, zeros_d)
        b_i = params["enc_convs"][i][1]
        gm, bt = params["enc_bns"][i]
        w_next = params["enc_convs"][i + 1][0]
        hw = _tc(_make_tc_mid(n, npad), f32((n, hid), _F32),
                 partials, dinv, r1(b_i), r1(gm), r1(bt), w_next)

    partials = prop(hw, packed_p, zeros_d)
    b3 = params["enc_convs"][3][1]
    gm3, bt3 = params["enc_bns"][3]
    (wl1, bl1), (wl2, bl2) = params["fc_latent"]
    (v1, c1), (v2, c2) = params["fc_decoder"]
    wd0 = params["dec_convs"][0][0]
    emb_dim = wl2.shape[1]
    hw, emb = _tc(_make_tc_bottleneck(n, npad, g),
                  [f32((n, hid), _F32), f32((g, emb_dim), _F32)],
                  partials, dinv, r1(b3), r1(gm3), r1(bt3), brow, bcol,
                  wl1, r1(bl1), wl2, r1(bl2), v1, r1(c1), v2, r1(c2), wd0)

    for i in range(3):
        partials = prop(hw, packed_p, zeros_d)
        b_i = params["dec_convs"][i][1]
        gm, bt = params["dec_bns"][i]
        w_next = params["dec_convs"][i + 1][0]
        hw = _tc(_make_tc_mid(n, npad), f32((n, hid), _F32),
                 partials, dinv, r1(b_i), r1(gm), r1(bt), w_next)

    partials = prop(hw, packed_p, zeros_d)
    b_last = params["dec_convs"][3][1]
    h_out = _tc(_make_tc_last(n, npad), f32((n, d_in), _F32),
                partials, dinv, r1(b_last))
    return (h_out, emb)
